# baseline (speedup 1.0000x reference)
import jax
import jax.numpy as jnp
from jax import lax
from jax.experimental import pallas as pl
from jax.experimental.pallas import tpu as pltpu

B = 4
N = 256
T = 256
IN = 32
H = 64
RES = 64
ND = T // RES
NC = 1
KP = (N * N) // 2


def _dot(a, b):
    return jnp.dot(a, b, preferred_element_type=jnp.float32)


def _fused_kernel(x_ref, wih_ref, whh_ref, bih_ref, bhh_ref, wq_ref, wk_ref,
                  wl1_ref, wr1_ref, wl2_ref, wr2_ref, clfw_ref, clfb_ref,
                  out_ref, h_state):
    w = pl.program_id(0)

    @pl.when(w == 0)
    def _init():
        h_state[...] = jnp.zeros((H, B * N), jnp.float32)

    wih = wih_ref[...].astype(jnp.bfloat16)
    whh = whh_ref[...].astype(jnp.bfloat16)
    bih = bih_ref[...]
    bhh = bhh_ref[...]

    def quad(qi, carry):
        h, s = carry
        xq = x_ref[:, pl.ds(qi * 4 * IN, 4 * IN)]
        xqT = xq.astype(jnp.bfloat16).T
        for j in range(4):
            xt = xqT[j * IN:(j + 1) * IN]
            gx = _dot(wih, xt) + bih
            gh = _dot(whh, h.astype(jnp.bfloat16)) + bhh
            r = jax.nn.sigmoid(gx[:H] + gh[:H])
            z = jax.nn.sigmoid(gx[H:2 * H] + gh[H:2 * H])
            n = jnp.tanh(gx[2 * H:] + r * gh[2 * H:])
            h = (1.0 - z) * n + z * h
            s = s + h
        return h, s

    h0 = h_state[...]
    s0 = jnp.zeros((H, B * N), jnp.float32)
    h_fin, s_fin = lax.fori_loop(0, RES // 4, quad, (h0, s0))
    h_state[...] = h_fin
    hpool = s_fin * (1.0 / RES)

    wq = wq_ref[...]
    wk = wk_ref[...]
    wl1 = wl1_ref[...]
    wr1 = wr1_ref[...]
    wl2 = wl2_ref[...]
    wr2 = wr2_ref[...]
    clfw = clfw_ref[...]

    ii = lax.broadcasted_iota(jnp.int32, (N, N), 0)
    jj = lax.broadcasted_iota(jnp.int32, (N, N), 1)
    diag = ii == jj

    hgs = []
    adjs = []
    for b in range(B):
        hg = hpool[:, b * N:(b + 1) * N]
        q = _dot(wq, hg)
        k = _dot(wk, hg)
        scores = lax.dot_general(q, k, (((0,), (0,)), ((), ())),
                                 preferred_element_type=jnp.float32) * 0.125
        m = jnp.max(scores, axis=-1, keepdims=True)
        e = jnp.exp(scores - m)
        attn = e / jnp.sum(e, axis=-1, keepdims=True)
        hgs.append(hg)
        adjs.append((attn + attn.T) * 0.5)

    def bs(_, st):
        nxt = []
        for b in range(B):
            lo, hi = st[2 * b], st[2 * b + 1]
            mid = lo + (hi - lo + 1) // 2
            v = lax.bitcast_convert_type(mid, jnp.float32)
            cnt = jnp.sum((adjs[b] >= v).astype(jnp.float32))
            big = cnt >= float(KP + 1)
            nxt += [jnp.where(big, mid, lo), jnp.where(big, hi, mid - 1)]
        return tuple(nxt)

    st = lax.fori_loop(0, 31, bs,
                       tuple([jnp.int32(0), jnp.int32(0x40000000)] * B))

    logits = []
    for b in range(B):
        hg = hgs[b]
        thr = lax.bitcast_convert_type(st[2 * b], jnp.float32)
        adj = adjs[b]
        adj = adj * (adj > thr).astype(jnp.float32)
        adj = jnp.where(diag, 1.0, adj)

        inv_deg = 1.0 / jnp.clip(jnp.sum(adj, axis=0, keepdims=True),
                                 1e-6, None)
        agg1 = _dot(hg, adj) * inv_deg
        h1 = jax.nn.relu(_dot(wl1, hg) + _dot(wr1, agg1))
        agg2 = _dot(h1, adj) * inv_deg
        h2 = jax.nn.relu(_dot(wl2, h1) + _dot(wr2, agg2))

        contrib = jnp.sum(h2, axis=1, keepdims=True) * (1.0 / ND)
        logits.append(_dot(clfw, contrib))

    contribs = jnp.concatenate(logits, axis=0)

    @pl.when(w == 0)
    def _first():
        out_ref[...] = contribs + clfb_ref[...]

    @pl.when(w > 0)
    def _rest():
        out_ref[...] = out_ref[...] + contribs


def kernel(x, batch_idx, W_ih, W_hh, b_ih, b_hh, W_q, W_k, W_l1, W_r1,
           W_l2, W_r2, clf_W, clf_b):
    del batch_idx
    x2d = x.reshape(B * N, T * IN)
    full = lambda a: pl.BlockSpec(a.shape, lambda w: (0,) * a.ndim)
    args = [W_ih.T, W_hh.T, b_ih.reshape(3 * H, 1), b_hh.reshape(3 * H, 1),
            W_q.T, W_k.T, W_l1.T, W_r1.T, W_l2.T, W_r2.T,
            clf_W.T, clf_b.reshape(1, NC)]

    out = pl.pallas_call(
        _fused_kernel,
        grid=(ND,),
        in_specs=[pl.BlockSpec((B * N, RES * IN), lambda w: (0, w))]
                 + [full(a) for a in args],
        out_specs=pl.BlockSpec((B, NC), lambda w: (0, 0)),
        out_shape=jax.ShapeDtypeStruct((B, NC), jnp.float32),
        scratch_shapes=[pltpu.VMEM((H, B * N), jnp.float32)],
        compiler_params=pltpu.CompilerParams(
            dimension_semantics=("arbitrary",)),
    )(x2d, *args)
    return out


# speedup vs baseline: 1.0384x; 1.0384x over previous
import jax
import jax.numpy as jnp
from jax import lax
from jax.experimental import pallas as pl
from jax.experimental.pallas import tpu as pltpu

B = 4
N = 256
T = 256
IN = 32
H = 64
RES = 64
ND = T // RES
NC = 1
KP = (N * N) // 2


def _dot(a, b):
    return jnp.dot(a, b, preferred_element_type=jnp.float32)


def _sig(x):
    return jnp.tanh(x * 0.5) * 0.5 + 0.5


def _fused_kernel(x_ref, wall_ref, wq_ref, wk_ref, wl1_ref, wr1_ref,
                  wl2_ref, wr2_ref, clfw_ref, clfb_ref, out_ref, h_state):
    w = pl.program_id(0)

    @pl.when(w == 0)
    def _init():
        h_state[...] = jnp.zeros((H, B * N), jnp.float32)

    wall = wall_ref[...]
    ones = jnp.ones((1, B * N), jnp.float32)

    def quad(qi, carry):
        h, s = carry
        xq = x_ref[:, pl.ds(qi * 4 * IN, 4 * IN)]
        xqT = xq.T
        for j in range(4):
            xt = xqT[j * IN:(j + 1) * IN]
            xh = jnp.concatenate([xt, h, ones], axis=0)
            g = _dot(wall, xh)
            r = _sig(g[:H])
            z = _sig(g[H:2 * H])
            n = jnp.tanh(g[2 * H:3 * H] + r * g[3 * H:])
            h = n + z * (h - n)
            s = s + h
        return h, s

    h0 = h_state[...]
    s0 = jnp.zeros((H, B * N), jnp.float32)
    h_fin, s_fin = lax.fori_loop(0, RES // 4, quad, (h0, s0))
    h_state[...] = h_fin
    hpool = s_fin * (1.0 / RES)

    wq = wq_ref[...]
    wk = wk_ref[...]
    wl1 = wl1_ref[...]
    wr1 = wr1_ref[...]
    wl2 = wl2_ref[...]
    wr2 = wr2_ref[...]
    clfw = clfw_ref[...]

    ii = lax.broadcasted_iota(jnp.int32, (N, N), 0)
    jj = lax.broadcasted_iota(jnp.int32, (N, N), 1)
    diag = ii == jj

    hgs = []
    adjs = []
    for b in range(B):
        hg = hpool[:, b * N:(b + 1) * N]
        q = _dot(wq, hg)
        k = _dot(wk, hg)
        scores = lax.dot_general(q, k, (((0,), (0,)), ((), ())),
                                 preferred_element_type=jnp.float32) * 0.125
        m = jnp.max(scores, axis=-1, keepdims=True)
        e = jnp.exp(scores - m)
        attn = e / jnp.sum(e, axis=-1, keepdims=True)
        hgs.append(hg)
        adjs.append((attn + attn.T) * 0.5)

    def bs(_, st):
        nxt = []
        for b in range(B):
            lo, hi = st[2 * b], st[2 * b + 1]
            mid = lo + (hi - lo + 1) // 2
            v = lax.bitcast_convert_type(mid, jnp.float32)
            cnt = jnp.sum((adjs[b] >= v).astype(jnp.float32))
            big = cnt >= float(KP + 1)
            nxt += [jnp.where(big, mid, lo), jnp.where(big, hi, mid - 1)]
        return tuple(nxt)

    st = lax.fori_loop(0, 31, bs,
                       tuple([jnp.int32(0), jnp.int32(0x40000000)] * B))

    logits = []
    for b in range(B):
        hg = hgs[b]
        thr = lax.bitcast_convert_type(st[2 * b], jnp.float32)
        adj = adjs[b]
        adj = adj * (adj > thr).astype(jnp.float32)
        adj = jnp.where(diag, 1.0, adj)

        inv_deg = 1.0 / jnp.clip(jnp.sum(adj, axis=0, keepdims=True),
                                 1e-6, None)
        agg1 = _dot(hg, adj) * inv_deg
        h1 = jax.nn.relu(_dot(wl1, hg) + _dot(wr1, agg1))
        agg2 = _dot(h1, adj) * inv_deg
        h2 = jax.nn.relu(_dot(wl2, h1) + _dot(wr2, agg2))

        contrib = jnp.sum(h2, axis=1, keepdims=True) * (1.0 / ND)
        logits.append(_dot(clfw, contrib))

    contribs = jnp.concatenate(logits, axis=0)

    @pl.when(w == 0)
    def _first():
        out_ref[...] = contribs + clfb_ref[...]

    @pl.when(w > 0)
    def _rest():
        out_ref[...] = out_ref[...] + contribs


def kernel(x, batch_idx, W_ih, W_hh, b_ih, b_hh, W_q, W_k, W_l1, W_r1,
           W_l2, W_r2, clf_W, clf_b):
    del batch_idx
    x2d = x.reshape(B * N, T * IN)
    full = lambda a: pl.BlockSpec(a.shape, lambda w: (0,) * a.ndim)
    wihT = W_ih.T
    whhT = W_hh.T
    zx = jnp.zeros((H, IN), jnp.float32)
    zh = jnp.zeros((H, H), jnp.float32)
    wall = jnp.concatenate([
        jnp.concatenate([wihT[:2 * H], whhT[:2 * H],
                         (b_ih + b_hh)[:2 * H].reshape(2 * H, 1)], axis=1),
        jnp.concatenate([wihT[2 * H:], zh,
                         b_ih[2 * H:].reshape(H, 1)], axis=1),
        jnp.concatenate([zx, whhT[2 * H:],
                         b_hh[2 * H:].reshape(H, 1)], axis=1),
    ], axis=0)
    args = [wall, W_q.T, W_k.T, W_l1.T, W_r1.T, W_l2.T, W_r2.T,
            clf_W.T, clf_b.reshape(1, NC)]

    out = pl.pallas_call(
        _fused_kernel,
        grid=(ND,),
        in_specs=[pl.BlockSpec((B * N, RES * IN), lambda w: (0, w))]
                 + [full(a) for a in args],
        out_specs=pl.BlockSpec((B, NC), lambda w: (0, 0)),
        out_shape=jax.ShapeDtypeStruct((B, NC), jnp.float32),
        scratch_shapes=[pltpu.VMEM((H, B * N), jnp.float32)],
        compiler_params=pltpu.CompilerParams(
            dimension_semantics=("arbitrary",)),
    )(x2d, *args)
    return out
